# baseline (speedup 1.0000x reference)
import functools

import jax
import jax.numpy as jnp
from jax import lax
from jax.experimental import pallas as pl
from jax.experimental.pallas import tpu as pltpu
from jax.experimental.pallas import tpu_sc as plsc

NUM_EXPERTS = 64
TOP_K = 2
BLOCK_M = 1024
NUM_TOKENS = 16384

_SC_INFO = plsc.get_sparse_core_info()
_NC, _NS, _L = _SC_INFO.num_cores, _SC_INFO.num_subcores, _SC_INFO.num_lanes
_NW = _NC * _NS
_TOK_PER_W = NUM_TOKENS // _NW
_GROUPS = _TOK_PER_W // _L

_MAGN = 0x7FFFFFFF


_EGROUPS = 2
_NCAND = 2 * _EGROUPS
_NEG = -3.4e38


def _matmul_block(x_ref, w_ref, logits_ref, vals_ref, ids_ref):
    l = jnp.dot(x_ref[...], w_ref[...], preferred_element_type=jnp.float32)
    logits_ref[...] = l
    gw = NUM_EXPERTS // _EGROUPS
    vcols, icols = [], []
    for g in range(_EGROUPS):
        lg = l[:, g * gw:(g + 1) * gw]
        eg = lax.broadcasted_iota(jnp.int32, lg.shape, 1) + g * gw
        m1 = jnp.max(lg, axis=1, keepdims=True)
        i1 = jnp.min(jnp.where(lg == m1, eg, NUM_EXPERTS), axis=1,
                     keepdims=True)
        lm = jnp.where(eg == i1, _NEG, lg)
        m2 = jnp.max(lm, axis=1, keepdims=True)
        i2 = jnp.min(jnp.where(lm == m2, eg, NUM_EXPERTS), axis=1,
                     keepdims=True)
        vcols += [m1, m2]
        icols += [i1, i2]
    vals = jnp.concatenate(vcols, axis=1)
    ids = jnp.concatenate(icols, axis=1)
    vals_ref[...] = jnp.swapaxes(
        vals.T.reshape(_NCAND, BLOCK_M // _TOK_PER_W, _TOK_PER_W), 0, 1)
    ids_ref[...] = jnp.swapaxes(
        ids.T.reshape(_NCAND, BLOCK_M // _TOK_PER_W, _TOK_PER_W), 0, 1)


def _tc_matmul(x, W):
    n, d = x.shape
    num_e = W.shape[1]
    return pl.pallas_call(
        _matmul_block,
        grid=(n // BLOCK_M,),
        in_specs=[
            pl.BlockSpec((BLOCK_M, d), lambda i: (i, 0)),
            pl.BlockSpec((d, num_e), lambda i: (0, 0)),
        ],
        out_specs=[
            pl.BlockSpec((BLOCK_M, num_e), lambda i: (i, 0)),
            pl.BlockSpec((BLOCK_M // _TOK_PER_W, _NCAND, _TOK_PER_W),
                         lambda i: (i, 0, 0)),
            pl.BlockSpec((BLOCK_M // _TOK_PER_W, _NCAND, _TOK_PER_W),
                         lambda i: (i, 0, 0)),
        ],
        out_shape=[
            jax.ShapeDtypeStruct((n, num_e), jnp.float32),
            jax.ShapeDtypeStruct((_NW, _NCAND, _TOK_PER_W), jnp.float32),
            jax.ShapeDtypeStruct((_NW, _NCAND, _TOK_PER_W), jnp.int32),
        ],
        compiler_params=pltpu.CompilerParams(
            dimension_semantics=("parallel",),
        ),
    )(x, W)


def _sc_router(vals_hbm, cids_hbm, ids1_hbm, ids2_hbm, p1_hbm, p2_hbm,
               vals_v, cids_v, i1_v, i2_v, p1_v, p2_v):
    wid = lax.axis_index("s") * _NC + lax.axis_index("c")
    base = wid * _TOK_PER_W
    pltpu.sync_copy(vals_hbm.at[wid], vals_v)
    pltpu.sync_copy(cids_hbm.at[wid], cids_v)

    neg = jnp.full((_L,), -3.4e38, jnp.float32)
    zero_i = jnp.zeros((_L,), jnp.int32)

    def group_body(g, carry):
        m1, i1, m2, i2 = neg, zero_i, neg, zero_i
        col = g * _L
        for c in range(_NCAND):
            v = vals_v[c, pl.ds(col, _L)]
            cid = cids_v[c, pl.ds(col, _L)]
            gt1 = v > m1
            gt2 = v > m2
            m2 = jnp.where(gt1, m1, jnp.where(gt2, v, m2))
            i2 = jnp.where(gt1, i1, jnp.where(gt2, cid, i2))
            m1 = jnp.where(gt1, v, m1)
            i1 = jnp.where(gt1, cid, i1)
        i1_v[pl.ds(col, _L)] = i1
        i2_v[pl.ds(col, _L)] = i2
        e2 = jnp.exp(m2 - m1)
        p1 = 1.0 / (1.0 + e2)
        p1_v[pl.ds(col, _L)] = p1
        p2_v[pl.ds(col, _L)] = 1.0 - p1
        return carry

    lax.fori_loop(0, _GROUPS, group_body, 0)

    pltpu.sync_copy(i1_v, ids1_hbm.at[pl.ds(base, _TOK_PER_W)])
    pltpu.sync_copy(i2_v, ids2_hbm.at[pl.ds(base, _TOK_PER_W)])
    pltpu.sync_copy(p1_v, p1_hbm.at[pl.ds(base, _TOK_PER_W)])
    pltpu.sync_copy(p2_v, p2_hbm.at[pl.ds(base, _TOK_PER_W)])


_sc_router_call = functools.partial(
    pl.kernel,
    mesh=plsc.VectorSubcoreMesh(core_axis_name="c", subcore_axis_name="s"),
    out_type=[
        jax.ShapeDtypeStruct((NUM_TOKENS,), jnp.int32),
        jax.ShapeDtypeStruct((NUM_TOKENS,), jnp.int32),
        jax.ShapeDtypeStruct((NUM_TOKENS,), jnp.float32),
        jax.ShapeDtypeStruct((NUM_TOKENS,), jnp.float32),
    ],
    scratch_types=[
        pltpu.VMEM((_NCAND, _TOK_PER_W), jnp.float32),
        pltpu.VMEM((_NCAND, _TOK_PER_W), jnp.int32),
        pltpu.VMEM((_TOK_PER_W,), jnp.int32),
        pltpu.VMEM((_TOK_PER_W,), jnp.int32),
        pltpu.VMEM((_TOK_PER_W,), jnp.float32),
        pltpu.VMEM((_TOK_PER_W,), jnp.float32),
    ],
)(_sc_router)


@jax.jit
def kernel(x, W):
    if x.ndim == 3:
        x = x.reshape(-1, x.shape[-1])
    logits, cand_vals, cand_ids = _tc_matmul(x, W)
    ids1, ids2, p1, p2 = _sc_router_call(cand_vals, cand_ids)
    ids = jnp.stack([ids1, ids2], axis=-1)
    probs = jnp.stack([p1, p2], axis=-1)
    return ids, probs, logits


# speedup vs baseline: 1.0446x; 1.0446x over previous
import functools

import jax
import jax.numpy as jnp
from jax import lax
from jax.experimental import pallas as pl
from jax.experimental.pallas import tpu as pltpu
from jax.experimental.pallas import tpu_sc as plsc

NUM_EXPERTS = 64
TOP_K = 2
BLOCK_M = 1024
NUM_TOKENS = 16384

_SC_INFO = plsc.get_sparse_core_info()
_NC, _NS, _L = _SC_INFO.num_cores, _SC_INFO.num_subcores, _SC_INFO.num_lanes
_NW = _NC * _NS
_TOK_PER_W = NUM_TOKENS // _NW
_GROUPS = _TOK_PER_W // _L

_NEG = -3.4e38


def _matmul_block(x_ref, w_ref, logits_ref, ids_ref, gap_ref):
    l = jnp.dot(x_ref[...], w_ref[...], preferred_element_type=jnp.float32)
    logits_ref[...] = l
    e = lax.broadcasted_iota(jnp.int32, l.shape, 1)
    m1 = jnp.max(l, axis=-1, keepdims=True)
    i1 = jnp.min(jnp.where(l == m1, e, NUM_EXPERTS), axis=-1, keepdims=True)
    lm = jnp.where(e == i1, _NEG, l)
    m2 = jnp.max(lm, axis=-1, keepdims=True)
    i2 = jnp.min(jnp.where(lm == m2, e, NUM_EXPERTS), axis=-1, keepdims=True)
    ids_ref[...] = jnp.concatenate([i1, i2], axis=-1)
    gap_ref[...] = (m2 - m1).reshape(1, BLOCK_M // _TOK_PER_W, _TOK_PER_W)


def _tc_matmul(x, W):
    n, d = x.shape
    num_e = W.shape[1]
    return pl.pallas_call(
        _matmul_block,
        grid=(n // BLOCK_M,),
        in_specs=[
            pl.BlockSpec((BLOCK_M, d), lambda i: (i, 0)),
            pl.BlockSpec((d, num_e), lambda i: (0, 0)),
        ],
        out_specs=[
            pl.BlockSpec((BLOCK_M, num_e), lambda i: (i, 0)),
            pl.BlockSpec((BLOCK_M, TOP_K), lambda i: (i, 0)),
            pl.BlockSpec((1, BLOCK_M // _TOK_PER_W, _TOK_PER_W),
                         lambda i: (i, 0, 0)),
        ],
        out_shape=[
            jax.ShapeDtypeStruct((n, num_e), jnp.float32),
            jax.ShapeDtypeStruct((n, TOP_K), jnp.int32),
            jax.ShapeDtypeStruct(
                (_NW // (BLOCK_M // _TOK_PER_W), BLOCK_M // _TOK_PER_W,
                 _TOK_PER_W), jnp.float32),
        ],
        compiler_params=pltpu.CompilerParams(
            dimension_semantics=("parallel",),
        ),
    )(x, W)


def _sc_router(gap_hbm, p1_hbm, p2_hbm, gap_v, p1_v, p2_v):
    wid = lax.axis_index("s") * _NC + lax.axis_index("c")
    base = wid * _TOK_PER_W
    sub = BLOCK_M // _TOK_PER_W
    pltpu.sync_copy(gap_hbm.at[wid // sub, wid % sub], gap_v)

    def group_body(g, carry):
        col = g * _L
        d = gap_v[pl.ds(col, _L)]
        p1 = 1.0 / (1.0 + jnp.exp(d))
        p1_v[pl.ds(col, _L)] = p1
        p2_v[pl.ds(col, _L)] = 1.0 - p1
        return carry

    lax.fori_loop(0, _GROUPS, group_body, 0)

    pltpu.sync_copy(p1_v, p1_hbm.at[pl.ds(base, _TOK_PER_W)])
    pltpu.sync_copy(p2_v, p2_hbm.at[pl.ds(base, _TOK_PER_W)])


_sc_router_call = functools.partial(
    pl.kernel,
    mesh=plsc.VectorSubcoreMesh(core_axis_name="c", subcore_axis_name="s"),
    out_type=[
        jax.ShapeDtypeStruct((NUM_TOKENS,), jnp.float32),
        jax.ShapeDtypeStruct((NUM_TOKENS,), jnp.float32),
    ],
    scratch_types=[
        pltpu.VMEM((_TOK_PER_W,), jnp.float32),
        pltpu.VMEM((_TOK_PER_W,), jnp.float32),
        pltpu.VMEM((_TOK_PER_W,), jnp.float32),
    ],
)(_sc_router)


@jax.jit
def kernel(x, W):
    if x.ndim == 3:
        x = x.reshape(-1, x.shape[-1])
    logits, ids, gap = _tc_matmul(x, W)
    p1, p2 = _sc_router_call(gap)
    probs = jnp.stack([p1, p2], axis=-1)
    return ids, probs, logits


# speedup vs baseline: 1.1155x; 1.0679x over previous
import functools

import jax
import jax.numpy as jnp
from jax import lax
from jax.experimental import pallas as pl
from jax.experimental.pallas import tpu as pltpu
from jax.experimental.pallas import tpu_sc as plsc

NUM_EXPERTS = 64
TOP_K = 2
BLOCK_M = 1024
NUM_TOKENS = 16384

_SC_INFO = plsc.get_sparse_core_info()
_NC, _NS, _L = _SC_INFO.num_cores, _SC_INFO.num_subcores, _SC_INFO.num_lanes
_NW = _NC * _NS
_TOK_PER_W = NUM_TOKENS // _NW
_GROUPS = _TOK_PER_W // _L


def _matmul_block(x_ref, w_ref, logits_ref, logits_t_ref):
    l = jnp.dot(x_ref[...], w_ref[...], preferred_element_type=jnp.float32)
    logits_ref[...] = l
    logits_t_ref[...] = l.T


def _tc_matmul(x, W):
    n, d = x.shape
    num_e = W.shape[1]
    return pl.pallas_call(
        _matmul_block,
        grid=(n // BLOCK_M,),
        in_specs=[
            pl.BlockSpec((BLOCK_M, d), lambda i: (i, 0)),
            pl.BlockSpec((d, num_e), lambda i: (0, 0)),
        ],
        out_specs=[
            pl.BlockSpec((BLOCK_M, num_e), lambda i: (i, 0)),
            pl.BlockSpec((num_e, BLOCK_M), lambda i: (0, i)),
        ],
        out_shape=[
            jax.ShapeDtypeStruct((n, num_e), jnp.float32),
            jax.ShapeDtypeStruct((num_e, n), jnp.float32),
        ],
        compiler_params=pltpu.CompilerParams(
            dimension_semantics=("parallel",),
        ),
    )(x, W)


def _sc_router(logits_t_hbm, ids1_hbm, ids2_hbm, p1_hbm, p2_hbm,
               lt_v, i1_v, i2_v, p1_v, p2_v):
    wid = lax.axis_index("s") * _NC + lax.axis_index("c")
    base = wid * _TOK_PER_W
    pltpu.sync_copy(logits_t_hbm.at[:, pl.ds(base, _TOK_PER_W)], lt_v)

    neg = jnp.full((_L,), -3.4e38, jnp.float32)
    zero_i = jnp.zeros((_L,), jnp.int32)

    def group_body(g, carry):
        m1, i1, m2, i2 = neg, zero_i, neg, zero_i
        col = g * _L
        for e in range(NUM_EXPERTS):
            v = lt_v[e, pl.ds(col, _L)]
            e_vec = jnp.full((_L,), e, jnp.int32)
            gt1 = v > m1
            gt2 = v > m2
            m2 = jnp.where(gt1, m1, jnp.where(gt2, v, m2))
            i2 = jnp.where(gt1, i1, jnp.where(gt2, e_vec, i2))
            m1 = jnp.where(gt1, v, m1)
            i1 = jnp.where(gt1, e_vec, i1)
        i1_v[pl.ds(col, _L)] = i1
        i2_v[pl.ds(col, _L)] = i2
        e2 = jnp.exp(m2 - m1)
        p1 = 1.0 / (1.0 + e2)
        p1_v[pl.ds(col, _L)] = p1
        p2_v[pl.ds(col, _L)] = 1.0 - p1
        return carry

    lax.fori_loop(0, _GROUPS, group_body, 0)

    pltpu.sync_copy(i1_v, ids1_hbm.at[pl.ds(base, _TOK_PER_W)])
    pltpu.sync_copy(i2_v, ids2_hbm.at[pl.ds(base, _TOK_PER_W)])
    pltpu.sync_copy(p1_v, p1_hbm.at[pl.ds(base, _TOK_PER_W)])
    pltpu.sync_copy(p2_v, p2_hbm.at[pl.ds(base, _TOK_PER_W)])


_sc_router_call = functools.partial(
    pl.kernel,
    mesh=plsc.VectorSubcoreMesh(core_axis_name="c", subcore_axis_name="s"),
    out_type=[
        jax.ShapeDtypeStruct((NUM_TOKENS,), jnp.int32),
        jax.ShapeDtypeStruct((NUM_TOKENS,), jnp.int32),
        jax.ShapeDtypeStruct((NUM_TOKENS,), jnp.float32),
        jax.ShapeDtypeStruct((NUM_TOKENS,), jnp.float32),
    ],
    scratch_types=[
        pltpu.VMEM((NUM_EXPERTS, _TOK_PER_W), jnp.float32),
        pltpu.VMEM((_TOK_PER_W,), jnp.int32),
        pltpu.VMEM((_TOK_PER_W,), jnp.int32),
        pltpu.VMEM((_TOK_PER_W,), jnp.float32),
        pltpu.VMEM((_TOK_PER_W,), jnp.float32),
    ],
)(_sc_router)


@jax.jit
def kernel(x, W):
    if x.ndim == 3:
        x = x.reshape(-1, x.shape[-1])
    logits, logits_t = _tc_matmul(x, W)
    ids1, ids2, p1, p2 = _sc_router_call(logits_t)
    ids = jnp.stack([ids1, ids2], axis=-1)
    probs = jnp.stack([p1, p2], axis=-1)
    return ids, probs, logits
